# baseline (speedup 1.0000x reference)
import jax
import jax.numpy as jnp
from jax.experimental import pallas as pl
from jax.experimental.pallas import tpu as pltpu

_DECAY = 1.0
_TM = 128


def _infectivity_kernel(ti_ref, tjs_ref, cjs_ref, emb_ref, out_ref,
                        gt_ref, hist_ref):
    @pl.when(pl.program_id(0) == 0)
    def _prologue():
        gt_ref[:] = jnp.exp(_DECAY * (tjs_ref[:] - ti_ref[:]))
        hist_ref[:] = cjs_ref[0].astype(jnp.float32)

    phi_c = jax.lax.dot_general(
        hist_ref[:], emb_ref[:], (((1,), (1,)), ((), ())),
        preferred_element_type=jnp.float32)
    out_ref[:] = jax.lax.dot_general(
        phi_c, gt_ref[:], (((0,), (1,)), ((), ())),
        preferred_element_type=jnp.float32)


def kernel(ti, tjs, ci, cjs, emb_weight):
    B = ti.shape[0]
    L = tjs.shape[1]
    N, D = emb_weight.shape
    grid = (pl.cdiv(N, _TM),)
    out2d = pl.pallas_call(
        _infectivity_kernel,
        grid=grid,
        in_specs=[
            pl.BlockSpec((B, 1), lambda i: (0, 0)),
            pl.BlockSpec((1, L), lambda i: (0, 0)),
            pl.BlockSpec((1, L, D), lambda i: (0, 0, 0)),
            pl.BlockSpec((_TM, D), lambda i: (i, 0)),
        ],
        out_specs=pl.BlockSpec((_TM, B), lambda i: (i, 0)),
        scratch_shapes=[
            pltpu.VMEM((B, L), jnp.float32),
            pltpu.VMEM((L, D), jnp.float32),
        ],
        out_shape=jax.ShapeDtypeStruct((N, B), jnp.float32),
    )(ti, tjs, cjs, emb_weight)
    return out2d[:, :, None]


# speedup vs baseline: 1.2634x; 1.2634x over previous
import jax
import jax.numpy as jnp
from jax.experimental import pallas as pl
from jax.experimental.pallas import tpu as pltpu

_DECAY = 1.0
_TM = 128


def _infectivity_kernel(ti_ref, tjs_ref, cjs_ref, emb_ref, out_ref,
                        s_ref, eti_ref):
    @pl.when(pl.program_id(0) == 0)
    def _prologue():
        hist = cjs_ref[0].astype(jnp.float32)
        e_tjs = jnp.exp(_DECAY * tjs_ref[:])
        s_ref[:] = jax.lax.dot_general(
            e_tjs, hist, (((1,), (0,)), ((), ())),
            preferred_element_type=jnp.float32)
        eti_ref[:] = jnp.exp(-_DECAY * ti_ref[:])

    w = jax.lax.dot_general(
        emb_ref[:], s_ref[:], (((1,), (1,)), ((), ())),
        preferred_element_type=jnp.float32)
    out_ref[:] = w * eti_ref[:]


def kernel(ti, tjs, ci, cjs, emb_weight):
    B = ti.shape[0]
    L = tjs.shape[1]
    N, D = emb_weight.shape
    ti_row = ti.reshape(1, B)
    grid = (pl.cdiv(N, _TM),)
    out2d = pl.pallas_call(
        _infectivity_kernel,
        grid=grid,
        in_specs=[
            pl.BlockSpec((1, B), lambda i: (0, 0)),
            pl.BlockSpec((1, L), lambda i: (0, 0)),
            pl.BlockSpec((1, L, D), lambda i: (0, 0, 0)),
            pl.BlockSpec((_TM, D), lambda i: (i, 0)),
        ],
        out_specs=pl.BlockSpec((_TM, B), lambda i: (i, 0)),
        scratch_shapes=[
            pltpu.VMEM((1, D), jnp.float32),
            pltpu.VMEM((1, B), jnp.float32),
        ],
        out_shape=jax.ShapeDtypeStruct((N, B), jnp.float32),
    )(ti_row, tjs, cjs, emb_weight)
    return out2d[:, :, None]


# speedup vs baseline: 1.4587x; 1.1545x over previous
import jax
import jax.numpy as jnp
from jax.experimental import pallas as pl
from jax.experimental.pallas import tpu as pltpu

_DECAY = 1.0
_TM = 256


def _infectivity_kernel(ti_ref, tjs_ref, cjs_ref, emb_ref, out_ref,
                        s_ref, eti_ref):
    @pl.when(pl.program_id(0) == 0)
    def _prologue():
        hist = cjs_ref[0].astype(jnp.float32)
        e_tjs = jnp.exp(_DECAY * tjs_ref[:])
        s_ref[:] = jax.lax.dot_general(
            e_tjs, hist, (((1,), (0,)), ((), ())),
            preferred_element_type=jnp.float32)
        eti_ref[:] = jnp.exp(-_DECAY * ti_ref[:])

    w = jax.lax.dot_general(
        emb_ref[:], s_ref[:], (((1,), (1,)), ((), ())),
        preferred_element_type=jnp.float32)
    out_ref[:] = w * eti_ref[:]


def kernel(ti, tjs, ci, cjs, emb_weight):
    B = ti.shape[0]
    L = tjs.shape[1]
    N, D = emb_weight.shape
    ti_row = ti.reshape(1, B)
    grid = (pl.cdiv(N, _TM),)
    out2d = pl.pallas_call(
        _infectivity_kernel,
        grid=grid,
        in_specs=[
            pl.BlockSpec((1, B), lambda i: (0, 0)),
            pl.BlockSpec((1, L), lambda i: (0, 0)),
            pl.BlockSpec((1, L, D), lambda i: (0, 0, 0)),
            pl.BlockSpec((_TM, D), lambda i: (i, 0)),
        ],
        out_specs=pl.BlockSpec((_TM, B), lambda i: (i, 0)),
        scratch_shapes=[
            pltpu.VMEM((1, D), jnp.float32),
            pltpu.VMEM((1, B), jnp.float32),
        ],
        out_shape=jax.ShapeDtypeStruct((N, B), jnp.float32),
    )(ti_row, tjs, cjs, emb_weight)
    return out2d[:, :, None]
